# baseline (speedup 1.0000x reference)
import math

import jax
import jax.numpy as jnp
from jax.experimental import pallas as pl


def kernel(x, pos_encoding):
    b, l, d = x.shape
    scale = math.sqrt(d)

    blk = 256
    while l % blk:
        blk //= 2
    nsb = l // blk

    pe = pos_encoding[:l] if pos_encoding.shape[0] != l else pos_encoding

    def body(x_ref, pe_ref, o_ref):
        i = pl.program_id(0)
        xv = x_ref[...]
        peb = pe_ref[pl.ds(i * blk, blk), :]
        o_ref[...] = jnp.where(xv == 0.0, 0.0, xv * scale + peb[None])

    return pl.pallas_call(
        body,
        grid=(nsb,),
        in_specs=[
            pl.BlockSpec((b, blk, d), lambda i: (0, i, 0)),
            pl.BlockSpec((l, d), lambda i: (0, 0)),
        ],
        out_specs=pl.BlockSpec((b, blk, d), lambda i: (0, i, 0)),
        out_shape=jax.ShapeDtypeStruct((b, l, d), x.dtype),
    )(x, pe)
